# baseline (speedup 1.0000x reference)
import functools

import jax
import jax.numpy as jnp
from jax import lax
from jax.experimental import pallas as pl
from jax.experimental.pallas import tpu as pltpu
from jax.experimental.pallas import tpu_sc as plsc

D = 64
N = 4096 * 50
NC, NS = 2, 16
NW = NC * NS
PER_W = N // NW
CHUNK = 128
NBUF = 2
NCHUNK = PER_W // CHUNK
ROUNDS = NCHUNK // NBUF
GROUPS = CHUNK // 16

_MESH = plsc.VectorSubcoreMesh(core_axis_name="c", subcore_axis_name="s")


@functools.partial(
    pl.kernel,
    mesh=_MESH,
    out_type=jax.ShapeDtypeStruct((N // 2, 2 * D), jnp.float32),
    scratch_types=[
        pltpu.VMEM((PER_W,), jnp.int32),
        pltpu.VMEM((PER_W,), jnp.int32),
        pltpu.VMEM((NBUF, CHUNK, 2 * D), jnp.float32),
        pltpu.VMEM((NBUF, CHUNK // 2, 2 * D), jnp.float32),
        pltpu.SemaphoreType.DMA((NBUF,)),
        pltpu.SemaphoreType.DMA((NBUF,)),
    ],
    compiler_params=pltpu.CompilerParams(
        use_tc_tiling_on_sc=True, needs_layout_passes=False),
)
def _gather_kernel(idx2_hbm, par_hbm, table_hbm, out_hbm,
                   idx2_v, par_v, pairs, outb, sem_g, sem_w):
    wid = lax.axis_index("s") * NC + lax.axis_index("c")
    base = wid * PER_W
    pltpu.sync_copy(idx2_hbm.at[pl.ds(base, PER_W)], idx2_v)
    pltpu.sync_copy(par_hbm.at[pl.ds(base, PER_W)], par_v)

    def gather(b, c):
        return pltpu.make_async_copy(
            table_hbm.at[idx2_v.at[pl.ds(c * CHUNK, CHUNK)]],
            pairs.at[b], sem_g.at[b])

    def write(b, c):
        off = pl.multiple_of((base + c * CHUNK) // 2, CHUNK // 2)
        return pltpu.make_async_copy(
            outb.at[b],
            out_hbm.at[pl.ds(off, CHUNK // 2)],
            sem_w.at[b])

    iota = lax.iota(jnp.int32, 16)
    row_l = lax.shift_right_logical(iota, 1)
    col_out0 = lax.mul(lax.rem(iota, 2), 64)

    def compact(b, c):
        def group_body(g, carry):
            kl = g * 16 + iota
            pv = par_v[pl.ds(c * CHUNK + g * 16, 16)]
            colb = pv * 64
            rowv = g * 8 + row_l
            for cc in range(D):
                v = plsc.load_gather(pairs.at[b], [kl, colb + cc])
                plsc.store_scatter(outb.at[b], [rowv, col_out0 + cc], v)
            return carry
        lax.fori_loop(0, GROUPS, group_body, 0)

    for b in range(NBUF):
        gather(b, b).start()

    def round_body(r, carry):
        c0 = r * NBUF
        for b in range(NBUF):
            gather(b, c0 + b).wait()
            compact(b, c0 + b)
            write(b, c0 + b).start()
        for b in range(NBUF):
            write(b, c0 + b).wait()
            gather(b, c0 + NBUF + b).start()
        return carry

    lax.fori_loop(0, ROUNDS - 1, round_body, 0)

    c0 = (ROUNDS - 1) * NBUF
    for b in range(NBUF):
        gather(b, c0 + b).wait()
        compact(b, c0 + b)
        write(b, c0 + b).start()
    for b in range(NBUF):
        write(b, c0 + b).wait()


def kernel(input, weight):
    flat = input.reshape(-1).astype(jnp.int32)
    idx2 = flat >> 1
    par = flat & 1
    w2 = weight.reshape(weight.shape[0] // 2, 2 * D)
    out2 = _gather_kernel(idx2, par, w2)
    return out2.reshape(input.shape + (weight.shape[1],))


# speedup vs baseline: 1.2855x; 1.2855x over previous
import functools

import jax
import jax.numpy as jnp
from jax import lax
from jax.experimental import pallas as pl
from jax.experimental.pallas import tpu as pltpu
from jax.experimental.pallas import tpu_sc as plsc

D = 64
N = 4096 * 50
NC, NS = 2, 16
NW = NC * NS
PER_W = N // NW
CHUNK = 200
NBUF = 2
NCHUNK = PER_W // CHUNK
ROUNDS = NCHUNK // NBUF

_MESH = plsc.VectorSubcoreMesh(core_axis_name="c", subcore_axis_name="s")


@functools.partial(
    pl.kernel,
    mesh=_MESH,
    out_type=jax.ShapeDtypeStruct((N, 2 * D), jnp.float32),
    scratch_types=[
        pltpu.VMEM((PER_W,), jnp.int32),
        pltpu.VMEM((NBUF, CHUNK, 2 * D), jnp.float32),
        pltpu.SemaphoreType.DMA((NBUF,)),
        pltpu.SemaphoreType.DMA((NBUF,)),
    ],
    compiler_params=pltpu.CompilerParams(
        use_tc_tiling_on_sc=True, needs_layout_passes=False),
)
def _gather_kernel(idx2_hbm, table_hbm, out_hbm, idx2_v, pairs, sem_g, sem_w):
    wid = lax.axis_index("s") * NC + lax.axis_index("c")
    base = wid * PER_W
    pltpu.sync_copy(idx2_hbm.at[pl.ds(base, PER_W)], idx2_v)

    def gather(b, c):
        return pltpu.make_async_copy(
            table_hbm.at[idx2_v.at[pl.ds(c * CHUNK, CHUNK)]],
            pairs.at[b], sem_g.at[b])

    def write(b, c):
        off = pl.multiple_of(base + c * CHUNK, CHUNK)
        return pltpu.make_async_copy(
            pairs.at[b], out_hbm.at[pl.ds(off, CHUNK)], sem_w.at[b])

    for b in range(NBUF):
        gather(b, b).start()

    def round_body(r, carry):
        c0 = r * NBUF
        for b in range(NBUF):
            gather(b, c0 + b).wait()
            write(b, c0 + b).start()
        for b in range(NBUF):
            write(b, c0 + b).wait()
            gather(b, c0 + NBUF + b).start()
        return carry

    lax.fori_loop(0, ROUNDS - 1, round_body, 0)

    c0 = (ROUNDS - 1) * NBUF
    for b in range(NBUF):
        gather(b, c0 + b).wait()
        write(b, c0 + b).start()
    for b in range(NBUF):
        write(b, c0 + b).wait()


def kernel(input, weight):
    flat = input.reshape(-1).astype(jnp.int32)
    idx2 = flat >> 1
    par = flat & 1
    w2 = weight.reshape(weight.shape[0] // 2, 2 * D)
    pairs = _gather_kernel(idx2, w2)
    out = jnp.where((par == 1)[:, None], pairs[:, D:], pairs[:, :D])
    return out.reshape(input.shape + (weight.shape[1],))


# speedup vs baseline: 1.7783x; 1.3834x over previous
import functools

import jax
import jax.numpy as jnp
from jax import lax
from jax.experimental import pallas as pl
from jax.experimental.pallas import tpu as pltpu
from jax.experimental.pallas import tpu_sc as plsc

D = 64
V = 1000000
N = 4096 * 50

BLK_I = 4096
HB = BLK_I // 2
GRID = (V + BLK_I - 1) // BLK_I
W2_ROWS = GRID * HB


def _relayout_body(wt_ref, o_ref):
    x = wt_ref[...]
    o_ref[...] = jnp.concatenate([x[:, :HB].T, x[:, HB:].T], axis=1)


_relayout = pl.pallas_call(
    _relayout_body,
    grid=(GRID,),
    in_specs=[pl.BlockSpec((D, BLK_I), lambda i: (0, i))],
    out_specs=pl.BlockSpec((HB, 2 * D), lambda i: (i, 0)),
    out_shape=jax.ShapeDtypeStruct((W2_ROWS, 2 * D), jnp.float32),
)

NC, NS = 2, 16
NW = NC * NS
PER_W = N // NW
CHUNK = 200
NBUF = 2
NCHUNK = PER_W // CHUNK
ROUNDS = NCHUNK // NBUF

_MESH = plsc.VectorSubcoreMesh(core_axis_name="c", subcore_axis_name="s")


@functools.partial(
    pl.kernel,
    mesh=_MESH,
    out_type=jax.ShapeDtypeStruct((N, 2 * D), jnp.float32),
    scratch_types=[
        pltpu.VMEM((PER_W,), jnp.int32),
        pltpu.VMEM((NBUF, CHUNK, 2 * D), jnp.float32),
        pltpu.SemaphoreType.DMA((NBUF,)),
        pltpu.SemaphoreType.DMA((NBUF,)),
    ],
    compiler_params=pltpu.CompilerParams(
        use_tc_tiling_on_sc=True, needs_layout_passes=False),
)
def _gather_kernel(idx2_hbm, table_hbm, out_hbm, idx2_v, pairs, sem_g, sem_w):
    wid = lax.axis_index("s") * NC + lax.axis_index("c")
    base = wid * PER_W
    pltpu.sync_copy(idx2_hbm.at[pl.ds(base, PER_W)], idx2_v)

    def gather(b, c):
        return pltpu.make_async_copy(
            table_hbm.at[idx2_v.at[pl.ds(c * CHUNK, CHUNK)]],
            pairs.at[b], sem_g.at[b])

    def write(b, c):
        off = pl.multiple_of(base + c * CHUNK, CHUNK)
        return pltpu.make_async_copy(
            pairs.at[b], out_hbm.at[pl.ds(off, CHUNK)], sem_w.at[b])

    for b in range(NBUF):
        gather(b, b).start()

    def round_body(r, carry):
        c0 = r * NBUF
        for b in range(NBUF):
            gather(b, c0 + b).wait()
            write(b, c0 + b).start()
        for b in range(NBUF):
            write(b, c0 + b).wait()
            gather(b, c0 + NBUF + b).start()
        return carry

    lax.fori_loop(0, ROUNDS - 1, round_body, 0)

    c0 = (ROUNDS - 1) * NBUF
    for b in range(NBUF):
        gather(b, c0 + b).wait()
        write(b, c0 + b).start()
    for b in range(NBUF):
        write(b, c0 + b).wait()


def kernel(input, weight):
    flat = input.reshape(-1).astype(jnp.int32)
    idx2 = ((flat >> 12) << 11) + (flat & 2047)
    par = (flat >> 11) & 1
    w2 = _relayout(weight.T)
    pairs = _gather_kernel(idx2, w2)
    out = jnp.where((par == 1)[:, None], pairs[:, D:], pairs[:, :D])
    return out.reshape(input.shape + (weight.shape[1],))


# speedup vs baseline: 2.5523x; 1.4353x over previous
import functools

import jax
import jax.numpy as jnp
from jax import lax
from jax.experimental import pallas as pl
from jax.experimental.pallas import tpu as pltpu
from jax.experimental.pallas import tpu_sc as plsc

D = 64
V = 1000000
B, H = 4096, 50
N = B * H

BLK_I = 4096
HB = BLK_I // 2
GRID = (V + BLK_I - 1) // BLK_I
W2_ROWS = GRID * HB


def _relayout_body(wt_ref, o_ref):
    x = wt_ref[...]
    o_ref[...] = jnp.concatenate([x[:, :HB].T, x[:, HB:].T], axis=1)


_relayout = pl.pallas_call(
    _relayout_body,
    grid=(GRID,),
    in_specs=[pl.BlockSpec((D, BLK_I), lambda i: (0, i))],
    out_specs=pl.BlockSpec((HB, 2 * D), lambda i: (i, 0)),
    out_shape=jax.ShapeDtypeStruct((W2_ROWS, 2 * D), jnp.float32),
)

NC, NS = 2, 16
NW = NC * NS
PER_W = N // NW
CHUNK = 200
NBUF = 2
NCHUNK = PER_W // CHUNK
ROUNDS = NCHUNK // NBUF

_MESH = plsc.VectorSubcoreMesh(core_axis_name="c", subcore_axis_name="s")


@functools.partial(
    pl.kernel,
    mesh=_MESH,
    out_type=jax.ShapeDtypeStruct((N, 2 * D), jnp.float32),
    scratch_types=[
        pltpu.VMEM((PER_W,), jnp.int32),
        pltpu.VMEM((NBUF, CHUNK, 2 * D), jnp.float32),
        pltpu.SemaphoreType.DMA((NBUF,)),
        pltpu.SemaphoreType.DMA((NBUF,)),
    ],
    compiler_params=pltpu.CompilerParams(
        use_tc_tiling_on_sc=True, needs_layout_passes=False),
)
def _gather_kernel(idx2_hbm, table_hbm, out_hbm, idx2_v, pairs, sem_g, sem_w):
    wid = lax.axis_index("s") * NC + lax.axis_index("c")
    base = wid * PER_W
    pltpu.sync_copy(idx2_hbm.at[pl.ds(base, PER_W)], idx2_v)

    def gather(b, c):
        return pltpu.make_async_copy(
            table_hbm.at[idx2_v.at[pl.ds(c * CHUNK, CHUNK)]],
            pairs.at[b], sem_g.at[b])

    def write(b, c):
        off = pl.multiple_of(base + c * CHUNK, CHUNK)
        return pltpu.make_async_copy(
            pairs.at[b], out_hbm.at[pl.ds(off, CHUNK)], sem_w.at[b])

    for b in range(NBUF):
        gather(b, b).start()

    def round_body(r, carry):
        c0 = r * NBUF
        for b in range(NBUF):
            gather(b, c0 + b).wait()
            write(b, c0 + b).start()
        for b in range(NBUF):
            write(b, c0 + b).wait()
            gather(b, c0 + NBUF + b).start()
        return carry

    lax.fori_loop(0, ROUNDS - 1, round_body, 0)

    c0 = (ROUNDS - 1) * NBUF
    for b in range(NBUF):
        gather(b, c0 + b).wait()
        write(b, c0 + b).start()
    for b in range(NBUF):
        write(b, c0 + b).wait()


def _select_body(pairs_ref, par_ref, o_ref):
    xt = pairs_ref[...].T
    lo, hi = xt[:D, :], xt[D:, :]
    p = par_ref[0]
    o_ref[...] = (lo + (hi - lo) * p)[None]


_select_t = pl.pallas_call(
    _select_body,
    grid=(H,),
    in_specs=[
        pl.BlockSpec((B, 2 * D), lambda h: (h, 0)),
        pl.BlockSpec((1, 1, B), lambda h: (h, 0, 0)),
    ],
    out_specs=pl.BlockSpec((1, D, B), lambda h: (h, 0, 0)),
    out_shape=jax.ShapeDtypeStruct((H, D, B), jnp.float32),
)


def kernel(input, weight):
    flat = input.T.reshape(-1).astype(jnp.int32)
    idx2 = ((flat >> 12) << 11) + (flat & 2047)
    par_f = ((flat >> 11) & 1).astype(jnp.float32).reshape(H, 1, B)
    w2 = _relayout(weight.T)
    pairs = _gather_kernel(idx2, w2)
    out_t = _select_t(pairs.reshape(H * B, 2 * D), par_f)
    return jnp.transpose(out_t, (2, 0, 1))


# speedup vs baseline: 2.9578x; 1.1589x over previous
import functools

import jax
import jax.numpy as jnp
from jax import lax
from jax.experimental import pallas as pl
from jax.experimental.pallas import tpu as pltpu
from jax.experimental.pallas import tpu_sc as plsc

D = 64
V = 1000000
B, H = 4096, 50
N = B * H

BLK_I = 8192
HB = BLK_I // 2
GRID = (V + BLK_I - 1) // BLK_I
W2_ROWS = GRID * HB


def _relayout_body(wt_ref, o_ref):
    x = wt_ref[...]
    o_ref[...] = jnp.concatenate([x[:, :HB].T, x[:, HB:].T], axis=1)


_relayout = pl.pallas_call(
    _relayout_body,
    grid=(GRID,),
    in_specs=[pl.BlockSpec((D, BLK_I), lambda i: (0, i))],
    out_specs=pl.BlockSpec((HB, 2 * D), lambda i: (i, 0)),
    out_shape=jax.ShapeDtypeStruct((W2_ROWS, 2 * D), jnp.float32),
)

NC, NS = 2, 16
NW = NC * NS
PER_W = N // NW
CHUNK = 200
NBUF = 2
NCHUNK = PER_W // CHUNK
ROUNDS = NCHUNK // NBUF

_MESH = plsc.VectorSubcoreMesh(core_axis_name="c", subcore_axis_name="s")


@functools.partial(
    pl.kernel,
    mesh=_MESH,
    out_type=jax.ShapeDtypeStruct((N, 2 * D), jnp.float32),
    scratch_types=[
        pltpu.VMEM((PER_W,), jnp.int32),
        pltpu.VMEM((NBUF, CHUNK, 2 * D), jnp.float32),
        pltpu.SemaphoreType.DMA((NBUF,)),
        pltpu.SemaphoreType.DMA((NBUF,)),
    ],
    compiler_params=pltpu.CompilerParams(
        use_tc_tiling_on_sc=True, needs_layout_passes=False),
)
def _gather_kernel(idx2_hbm, table_hbm, out_hbm, idx2_v, pairs, sem_g, sem_w):
    wid = lax.axis_index("s") * NC + lax.axis_index("c")
    base = wid * PER_W
    pltpu.sync_copy(idx2_hbm.at[pl.ds(base, PER_W)], idx2_v)

    def gather(b, c):
        return pltpu.make_async_copy(
            table_hbm.at[idx2_v.at[pl.ds(c * CHUNK, CHUNK)]],
            pairs.at[b], sem_g.at[b])

    def write(b, c):
        off = pl.multiple_of(base + c * CHUNK, CHUNK)
        return pltpu.make_async_copy(
            pairs.at[b], out_hbm.at[pl.ds(off, CHUNK)], sem_w.at[b])

    for b in range(NBUF):
        gather(b, b).start()

    def round_body(r, carry):
        c0 = r * NBUF
        for b in range(NBUF):
            gather(b, c0 + b).wait()
            write(b, c0 + b).start()
        for b in range(NBUF):
            write(b, c0 + b).wait()
            gather(b, c0 + NBUF + b).start()
        return carry

    lax.fori_loop(0, ROUNDS - 1, round_body, 0)

    c0 = (ROUNDS - 1) * NBUF
    for b in range(NBUF):
        gather(b, c0 + b).wait()
        write(b, c0 + b).start()
    for b in range(NBUF):
        write(b, c0 + b).wait()


def _select_body(pairs_ref, par_ref, o_ref):
    xt = pairs_ref[...].T
    lo, hi = xt[:D, :], xt[D:, :]
    p = par_ref[0]
    o_ref[...] = (lo + (hi - lo) * p)[None]


_select_t = pl.pallas_call(
    _select_body,
    grid=(H,),
    in_specs=[
        pl.BlockSpec((B, 2 * D), lambda h: (h, 0)),
        pl.BlockSpec((1, 1, B), lambda h: (h, 0, 0)),
    ],
    out_specs=pl.BlockSpec((1, D, B), lambda h: (h, 0, 0)),
    out_shape=jax.ShapeDtypeStruct((H, D, B), jnp.float32),
)


def kernel(input, weight):
    flat = input.T.reshape(-1).astype(jnp.int32)
    sh_blk = BLK_I.bit_length() - 1
    sh_hb = HB.bit_length() - 1
    idx2 = ((flat >> sh_blk) << sh_hb) + (flat & (HB - 1))
    par_f = ((flat >> sh_hb) & 1).astype(jnp.float32).reshape(H, 1, B)
    w2 = _relayout(weight.T)
    pairs = _gather_kernel(idx2, w2)
    out_t = _select_t(pairs.reshape(H * B, 2 * D), par_f)
    return jnp.transpose(out_t, (2, 0, 1))


# speedup vs baseline: 3.1995x; 1.0817x over previous
import functools

import jax
import jax.numpy as jnp
from jax import lax
from jax.experimental import pallas as pl
from jax.experimental.pallas import tpu as pltpu
from jax.experimental.pallas import tpu_sc as plsc

D = 64
V = 1000000
B, H = 4096, 50
N = B * H

BLK_I = 16384
HB = BLK_I // 2
GRID = (V + BLK_I - 1) // BLK_I
W2_ROWS = GRID * HB


def _relayout_body(wt_ref, o_ref):
    x = wt_ref[...]
    o_ref[...] = jnp.concatenate([x[:, :HB].T, x[:, HB:].T], axis=1)


_relayout = pl.pallas_call(
    _relayout_body,
    grid=(GRID,),
    in_specs=[pl.BlockSpec((D, BLK_I), lambda i: (0, i))],
    out_specs=pl.BlockSpec((HB, 2 * D), lambda i: (i, 0)),
    out_shape=jax.ShapeDtypeStruct((W2_ROWS, 2 * D), jnp.float32),
)

NC, NS = 2, 16
NW = NC * NS
PER_W = N // NW
CHUNK = 200
NBUF = 2
NCHUNK = PER_W // CHUNK
ROUNDS = NCHUNK // NBUF

_MESH = plsc.VectorSubcoreMesh(core_axis_name="c", subcore_axis_name="s")


@functools.partial(
    pl.kernel,
    mesh=_MESH,
    out_type=jax.ShapeDtypeStruct((N, 2 * D), jnp.float32),
    scratch_types=[
        pltpu.VMEM((PER_W,), jnp.int32),
        pltpu.VMEM((NBUF, CHUNK, 2 * D), jnp.float32),
        pltpu.SemaphoreType.DMA((NBUF,)),
        pltpu.SemaphoreType.DMA((NBUF,)),
    ],
    compiler_params=pltpu.CompilerParams(
        use_tc_tiling_on_sc=True, needs_layout_passes=False),
)
def _gather_kernel(idx2_hbm, table_hbm, out_hbm, idx2_v, pairs, sem_g, sem_w):
    wid = lax.axis_index("s") * NC + lax.axis_index("c")
    base = wid * PER_W
    pltpu.sync_copy(idx2_hbm.at[pl.ds(base, PER_W)], idx2_v)

    def gather(b, c):
        return pltpu.make_async_copy(
            table_hbm.at[idx2_v.at[pl.ds(c * CHUNK, CHUNK)]],
            pairs.at[b], sem_g.at[b])

    def write(b, c):
        off = pl.multiple_of(base + c * CHUNK, CHUNK)
        return pltpu.make_async_copy(
            pairs.at[b], out_hbm.at[pl.ds(off, CHUNK)], sem_w.at[b])

    for b in range(NBUF):
        gather(b, b).start()

    def round_body(r, carry):
        c0 = r * NBUF
        for b in range(NBUF):
            gather(b, c0 + b).wait()
            write(b, c0 + b).start()
        for b in range(NBUF):
            write(b, c0 + b).wait()
            gather(b, c0 + NBUF + b).start()
        return carry

    lax.fori_loop(0, ROUNDS - 1, round_body, 0)

    c0 = (ROUNDS - 1) * NBUF
    for b in range(NBUF):
        gather(b, c0 + b).wait()
        write(b, c0 + b).start()
    for b in range(NBUF):
        write(b, c0 + b).wait()


def _select_body(pairs_ref, par_ref, o_ref):
    xt = pairs_ref[...].T
    lo, hi = xt[:D, :], xt[D:, :]
    p = par_ref[0]
    o_ref[...] = (lo + (hi - lo) * p)[None]


_select_t = pl.pallas_call(
    _select_body,
    grid=(H,),
    in_specs=[
        pl.BlockSpec((B, 2 * D), lambda h: (h, 0)),
        pl.BlockSpec((1, 1, B), lambda h: (h, 0, 0)),
    ],
    out_specs=pl.BlockSpec((1, D, B), lambda h: (h, 0, 0)),
    out_shape=jax.ShapeDtypeStruct((H, D, B), jnp.float32),
)


def kernel(input, weight):
    flat = input.T.reshape(-1).astype(jnp.int32)
    sh_blk = BLK_I.bit_length() - 1
    sh_hb = HB.bit_length() - 1
    idx2 = ((flat >> sh_blk) << sh_hb) + (flat & (HB - 1))
    par_f = ((flat >> sh_hb) & 1).astype(jnp.float32).reshape(H, 1, B)
    w2 = _relayout(weight.T)
    pairs = _gather_kernel(idx2, w2)
    out_t = _select_t(pairs.reshape(H * B, 2 * D), par_f)
    return jnp.transpose(out_t, (2, 0, 1))


# speedup vs baseline: 3.3256x; 1.0394x over previous
import functools

import jax
import jax.numpy as jnp
from jax import lax
from jax.experimental import pallas as pl
from jax.experimental.pallas import tpu as pltpu
from jax.experimental.pallas import tpu_sc as plsc

D = 64
V = 1000000
B, H = 4096, 50
N = B * H

BLK_I = 32768
HB = BLK_I // 2
GRID = (V + BLK_I - 1) // BLK_I
W2_ROWS = GRID * HB


def _relayout_body(wt_ref, o_ref):
    x = wt_ref[...]
    o_ref[...] = jnp.concatenate([x[:, :HB].T, x[:, HB:].T], axis=1)


_relayout = pl.pallas_call(
    _relayout_body,
    grid=(GRID,),
    in_specs=[pl.BlockSpec((D, BLK_I), lambda i: (0, i))],
    out_specs=pl.BlockSpec((HB, 2 * D), lambda i: (i, 0)),
    out_shape=jax.ShapeDtypeStruct((W2_ROWS, 2 * D), jnp.float32),
)

NC, NS = 2, 16
NW = NC * NS
PER_W = N // NW
CHUNK = 200
NBUF = 2
NCHUNK = PER_W // CHUNK
ROUNDS = NCHUNK // NBUF

_MESH = plsc.VectorSubcoreMesh(core_axis_name="c", subcore_axis_name="s")


@functools.partial(
    pl.kernel,
    mesh=_MESH,
    out_type=jax.ShapeDtypeStruct((N, 2 * D), jnp.float32),
    scratch_types=[
        pltpu.VMEM((PER_W,), jnp.int32),
        pltpu.VMEM((NBUF, CHUNK, 2 * D), jnp.float32),
        pltpu.SemaphoreType.DMA((NBUF,)),
        pltpu.SemaphoreType.DMA((NBUF,)),
    ],
    compiler_params=pltpu.CompilerParams(
        use_tc_tiling_on_sc=True, needs_layout_passes=False),
)
def _gather_kernel(idx2_hbm, table_hbm, out_hbm, idx2_v, pairs, sem_g, sem_w):
    wid = lax.axis_index("s") * NC + lax.axis_index("c")
    base = wid * PER_W
    pltpu.sync_copy(idx2_hbm.at[pl.ds(base, PER_W)], idx2_v)

    def gather(b, c):
        return pltpu.make_async_copy(
            table_hbm.at[idx2_v.at[pl.ds(c * CHUNK, CHUNK)]],
            pairs.at[b], sem_g.at[b])

    def write(b, c):
        off = pl.multiple_of(base + c * CHUNK, CHUNK)
        return pltpu.make_async_copy(
            pairs.at[b], out_hbm.at[pl.ds(off, CHUNK)], sem_w.at[b])

    for b in range(NBUF):
        gather(b, b).start()

    def round_body(r, carry):
        c0 = r * NBUF
        for b in range(NBUF):
            gather(b, c0 + b).wait()
            write(b, c0 + b).start()
        for b in range(NBUF):
            write(b, c0 + b).wait()
            gather(b, c0 + NBUF + b).start()
        return carry

    lax.fori_loop(0, ROUNDS - 1, round_body, 0)

    c0 = (ROUNDS - 1) * NBUF
    for b in range(NBUF):
        gather(b, c0 + b).wait()
        write(b, c0 + b).start()
    for b in range(NBUF):
        write(b, c0 + b).wait()


def _select_body(pairs_ref, par_ref, o_ref):
    xt = pairs_ref[...].T
    lo, hi = xt[:D, :], xt[D:, :]
    p = par_ref[0]
    o_ref[...] = (lo + (hi - lo) * p)[None]


_select_t = pl.pallas_call(
    _select_body,
    grid=(H,),
    in_specs=[
        pl.BlockSpec((B, 2 * D), lambda h: (h, 0)),
        pl.BlockSpec((1, 1, B), lambda h: (h, 0, 0)),
    ],
    out_specs=pl.BlockSpec((1, D, B), lambda h: (h, 0, 0)),
    out_shape=jax.ShapeDtypeStruct((H, D, B), jnp.float32),
)


def kernel(input, weight):
    flat = input.T.reshape(-1).astype(jnp.int32)
    sh_blk = BLK_I.bit_length() - 1
    sh_hb = HB.bit_length() - 1
    idx2 = ((flat >> sh_blk) << sh_hb) + (flat & (HB - 1))
    par_f = ((flat >> sh_hb) & 1).astype(jnp.float32).reshape(H, 1, B)
    w2 = _relayout(weight.T)
    pairs = _gather_kernel(idx2, w2)
    out_t = _select_t(pairs.reshape(H * B, 2 * D), par_f)
    return jnp.transpose(out_t, (2, 0, 1))
